# Initial kernel scaffold; baseline (speedup 1.0000x reference)
import jax
import jax.numpy as jnp
from jax.experimental import pallas as pl


def kernel(s1_data, in_lai, in_mask_lai, glob, W1, b1, Wg, W2, b2):
    raise NotImplementedError("write your pallas kernel here")


# speedup vs baseline: 2.3708x; 2.3708x over previous
import functools

import jax
import jax.numpy as jnp
from jax import lax
from jax.experimental import pallas as pl

_B, _T, _CS1, _G, _H, _W = 16, 6, 12, 8, 256, 256
_HID = 16
_CIN = _CS1 + 2 * _T

_ROWS = _H + 4
_CHUNKS = _W // 8
_XCROWS = _ROWS * _CHUNKS
_HROWS = (_H + 2) * _CHUNKS
_OROWS = _H * _CHUNKS


def _conv_kernel(xc_ref, glob_ref, wg_ref, b1_ref, bb1_ref, bb2_ref, b2_ref,
                 out_ref):
    g = jnp.dot(glob_ref[0], wg_ref[...],
                preferred_element_type=jnp.float32)
    bias16 = g + b1_ref[...]
    bias128 = jnp.tile(bias16, (1, 8))

    xc = xc_ref[0]

    h = jnp.zeros((_HROWS, 128), dtype=jnp.float32)
    for dy in range(3):
        h = h + lax.dot_general(
            xc[dy * _CHUNKS: dy * _CHUNKS + _HROWS], bb1_ref[dy],
            ((( 1,), (0,)), ((), ())),
            preferred_element_type=jnp.float32)
    h = jnp.maximum(h + bias128, 0.0)
    rall = lax.broadcasted_iota(jnp.int32, (_HROWS, 1), 0)
    h = jnp.where((rall >= _CHUNKS) & (rall < _HROWS - _CHUNKS), h, 0.0)

    zeros16 = jnp.zeros((1, 16), dtype=jnp.float32)
    left = jnp.concatenate([zeros16, h[:-1, 112:128]], axis=0)
    right = jnp.concatenate([h[1:, 0:16], zeros16], axis=0)
    ridx = lax.broadcasted_iota(jnp.int32, (_HROWS, 1), 0) % _CHUNKS
    left = jnp.where(ridx == 0, 0.0, left)
    right = jnp.where(ridx == _CHUNKS - 1, 0.0, right)
    hw = jnp.concatenate([left, h, right], axis=1)

    q = lax.dot_general(hw, bb2_ref[...], (((1,), (0,)), ((), ())),
                        preferred_element_type=jnp.float32)

    acc = q[0:_OROWS, 0:8]
    acc = acc + q[_CHUNKS:_CHUNKS + _OROWS, 8:16]
    acc = acc + q[2 * _CHUNKS:2 * _CHUNKS + _OROWS, 16:24]
    acc = acc + b2_ref[...]
    acc3 = acc.reshape(_H, _CHUNKS, 8)
    for k in range(_CHUNKS):
        out_ref[0, :, 8 * k: 8 * k + 8] = acc3[:, k, :]


def _band_w1(w1):
    wp = jnp.transpose(w1, (2, 3, 1, 0))
    b = jnp.zeros((3, 10, _CIN, 8, _HID), dtype=jnp.float32)
    j = jnp.arange(8)
    for dx in range(3):
        val = jnp.broadcast_to(wp[:, dx][None, :, :, :], (8, 3, _CIN, _HID))
        b = b.at[:, j + dx, :, j, :].set(val)
    return b.reshape(3, 240, 128)


def _band_w2(w2):
    wp = jnp.transpose(w2[0], (0, 1, 2))
    b = jnp.zeros((10, _HID, 3, 8), dtype=jnp.float32)
    j = jnp.arange(8)
    for dx in range(3):
        val = jnp.broadcast_to(wp[:, :, dx][None], (8, _HID, 3))
        b = b.at[j + dx, :, :, j].set(val)
    return b.reshape(160, 24)


@functools.partial(jax.jit, static_argnames=())
def kernel(s1_data, in_lai, in_mask_lai, glob, W1, b1, Wg, W2, b2):
    x = jnp.concatenate([s1_data, in_lai, in_mask_lai], axis=1)
    xt = jnp.transpose(x, (0, 2, 3, 1))
    xp = jnp.pad(xt, ((0, 0), (2, 2), (1, 7), (0, 0)))
    a = xp[:, :, 0:_W, :].reshape(_B, _ROWS, _CHUNKS, 8 * _CIN)
    c2 = xp[:, :, 8:8 + _W, :].reshape(_B, _ROWS, _CHUNKS, 8, _CIN)
    c2 = c2[:, :, :, 0:2, :].reshape(_B, _ROWS, _CHUNKS, 2 * _CIN)
    xc = jnp.concatenate([a, c2], axis=-1)
    xc = xc.reshape(_B, _XCROWS, 240)

    bb1 = _band_w1(W1)
    bb2 = _band_w2(W2)
    glob3 = glob.reshape(_B, 1, _G)

    out = pl.pallas_call(
        _conv_kernel,
        grid=(_B,),
        in_specs=[
            pl.BlockSpec((1, _XCROWS, 240), lambda b: (b, 0, 0)),
            pl.BlockSpec((1, 1, _G), lambda b: (b, 0, 0)),
            pl.BlockSpec((_G, _HID), lambda b: (0, 0)),
            pl.BlockSpec((1, _HID), lambda b: (0, 0)),
            pl.BlockSpec((3, 240, 128), lambda b: (0, 0, 0)),
            pl.BlockSpec((160, 24), lambda b: (0, 0)),
            pl.BlockSpec((1, 8), lambda b: (0, 0)),
        ],
        out_specs=pl.BlockSpec((1, _H, _W), lambda b: (b, 0, 0)),
        out_shape=jax.ShapeDtypeStruct((_B, _H, _W), jnp.float32),
    )(xc, glob3, Wg, b1.reshape(1, _HID), bb1, bb2,
      jnp.broadcast_to(b2.reshape(1, 1), (1, 8)))

    return (out.reshape(_B, 1, _H, _W),)


# speedup vs baseline: 17.4127x; 7.3445x over previous
import functools

import jax
import jax.numpy as jnp
from jax import lax
from jax.experimental import pallas as pl
from jax.experimental.pallas import tpu as pltpu

_B, _T, _CS1, _G, _H, _W = 16, 6, 12, 8, 256, 256
_HID = 16
_CIN = _CS1 + 2 * _T

_NP = _H * _W
_NF = (_H + 2) * _W
_NC = 16384
_NCHUNKS = _NP // _NC


def _conv_kernel(xf_ref, glob_ref, wg_ref, b1_ref, w1_ref, w2_ref, b2_ref,
                 out_ref, hz_ref):
    g = jnp.dot(glob_ref[0], wg_ref[...],
                preferred_element_type=jnp.float32)
    bias_col = jnp.transpose(g + b1_ref[...], (1, 0))

    lane = lax.broadcasted_iota(jnp.int32, (1, _NC), 1) % _W
    mask_l = lane == 0
    mask_r = lane == _W - 1

    hz_ref[:, 0:_W] = jnp.zeros((_HID, _W), jnp.float32)
    hz_ref[:, _NF - _W:_NF] = jnp.zeros((_HID, _W), jnp.float32)

    for c in range(_NCHUNKS):
        base = c * _NC
        a3 = jnp.concatenate(
            [xf_ref[0, :, base + dy * _W: base + dy * _W + _NC]
             for dy in range(3)], axis=0)
        p = lax.dot_general(w1_ref[...], a3, (((1,), (0,)), ((), ())),
                            preferred_element_type=jnp.float32)
        p0 = jnp.where(mask_l, 0.0, jnp.roll(p[0:16], 1, axis=1))
        p2 = jnp.where(mask_r, 0.0, jnp.roll(p[32:48], -1, axis=1))
        hc = p[16:32] + p0 + p2
        hc = jnp.maximum(hc + bias_col, 0.0)
        hz_ref[:, _W + base: _W + base + _NC] = hc

    for c in range(_NCHUNKS):
        base = c * _NC
        h3 = jnp.concatenate(
            [hz_ref[:, base + dy * _W: base + dy * _W + _NC]
             for dy in range(3)], axis=0)
        q = lax.dot_general(w2_ref[...], h3, (((1,), (0,)), ((), ())),
                            preferred_element_type=jnp.float32)
        q0 = jnp.where(mask_l, 0.0, jnp.roll(q[0:1], 1, axis=1))
        q2 = jnp.where(mask_r, 0.0, jnp.roll(q[2:3], -1, axis=1))
        oc = q[1:2] + q0 + q2 + b2_ref[0, 0]
        out_ref[0, 2 * c: 2 * c + 2, :] = jnp.concatenate(
            [oc[:, 0:_NC // 2], oc[:, _NC // 2:_NC]], axis=0)


@functools.partial(jax.jit, static_argnames=())
def kernel(s1_data, in_lai, in_mask_lai, glob, W1, b1, Wg, W2, b2):
    x = jnp.concatenate([s1_data, in_lai, in_mask_lai], axis=1)
    xf = jnp.pad(x, ((0, 0), (0, 0), (1, 1), (0, 0)))
    xf = xf.reshape(_B, _CIN, _NF)

    w1m = jnp.transpose(W1, (3, 0, 2, 1)).reshape(48, 72)
    w2m = jnp.transpose(W2[0], (2, 1, 0)).reshape(3, 48)

    out = pl.pallas_call(
        _conv_kernel,
        grid=(_B,),
        in_specs=[
            pl.BlockSpec((1, _CIN, _NF), lambda b: (b, 0, 0)),
            pl.BlockSpec((1, 1, _G), lambda b: (b, 0, 0)),
            pl.BlockSpec((_G, _HID), lambda b: (0, 0)),
            pl.BlockSpec((1, _HID), lambda b: (0, 0)),
            pl.BlockSpec((48, 72), lambda b: (0, 0)),
            pl.BlockSpec((3, 48), lambda b: (0, 0)),
            pl.BlockSpec((1, 1), lambda b: (0, 0)),
        ],
        out_specs=pl.BlockSpec((1, 8, _NP // 8), lambda b: (b, 0, 0)),
        out_shape=jax.ShapeDtypeStruct((_B, 8, _NP // 8), jnp.float32),
        scratch_shapes=[pltpu.VMEM((_HID, _NF), jnp.float32)],
    )(xf, glob.reshape(_B, 1, _G), Wg, b1.reshape(1, _HID), w1m, w2m,
      b2.reshape(1, 1))

    return (out.reshape(_B, 1, _H, _W),)


# speedup vs baseline: 21.5795x; 1.2393x over previous
import functools

import jax
import jax.numpy as jnp
from jax import lax
from jax.experimental import pallas as pl
from jax.experimental.pallas import tpu as pltpu

_B, _T, _CS1, _G, _H, _W = 16, 6, 12, 8, 256, 256
_HID = 16
_CIN = _CS1 + 2 * _T

_NP = _H * _W
_NF = (_H + 2) * _W
_NC = 16384
_NCHUNKS = _NP // _NC


def _dyslice(ref, nch, base, dy):
    lo = base + (dy - 1) * _W
    if lo < 0:
        return jnp.concatenate(
            [jnp.zeros((nch, -lo), jnp.float32), ref[0, :, 0:_NC + lo]],
            axis=1)
    if lo + _NC > _NP:
        return jnp.concatenate(
            [ref[0, :, lo:_NP], jnp.zeros((nch, lo + _NC - _NP), jnp.float32)],
            axis=1)
    return ref[0, :, lo:lo + _NC]


def _conv_kernel(s1_ref, lai_ref, msk_ref, glob_ref, wg_ref, b1_ref,
                 w1_ref, w2_ref, b2_ref, out_ref, hz_ref):
    g = jnp.dot(glob_ref[0], wg_ref[...],
                preferred_element_type=jnp.float32)
    bias_col = jnp.transpose(g + b1_ref[...], (1, 0))

    lane = lax.broadcasted_iota(jnp.int32, (1, _NC), 1) % _W
    mask_l = lane == 0
    mask_r = lane == _W - 1

    hz_ref[:, 0:_W] = jnp.zeros((_HID, _W), jnp.float32)
    hz_ref[:, _NF - _W:_NF] = jnp.zeros((_HID, _W), jnp.float32)

    zpad4 = jnp.zeros((4, _NC), jnp.float32)
    zpad2 = jnp.zeros((2, _NC), jnp.float32)

    for c in range(_NCHUNKS):
        base = c * _NC
        pieces = []
        for dy in range(3):
            pieces += [_dyslice(s1_ref, _CS1, base, dy), zpad4]
        for dy in range(3):
            pieces += [_dyslice(lai_ref, _T, base, dy), zpad2]
        for dy in range(3):
            pieces += [_dyslice(msk_ref, _T, base, dy), zpad2]
        a3 = jnp.concatenate(pieces, axis=0)
        p = lax.dot_general(w1_ref[...], a3, (((1,), (0,)), ((), ())),
                            preferred_element_type=jnp.float32)
        p0 = jnp.where(mask_l, 0.0, jnp.roll(p[0:16], 1, axis=1))
        p2 = jnp.where(mask_r, 0.0, jnp.roll(p[32:48], -1, axis=1))
        hc = p[16:32] + p0 + p2
        hc = jnp.maximum(hc + bias_col, 0.0)
        hz_ref[:, _W + base: _W + base + _NC] = hc

    for c in range(_NCHUNKS):
        base = c * _NC
        h3 = jnp.concatenate(
            [hz_ref[:, base + dy * _W: base + dy * _W + _NC]
             for dy in range(3)], axis=0)
        q = lax.dot_general(w2_ref[...], h3, (((1,), (0,)), ((), ())),
                            preferred_element_type=jnp.float32)
        q0 = jnp.where(mask_l, 0.0, jnp.roll(q[0:1], 1, axis=1))
        q2 = jnp.where(mask_r, 0.0, jnp.roll(q[2:3], -1, axis=1))
        oc = q[1:2] + q0 + q2 + b2_ref[0, 0]
        out_ref[0, 2 * c: 2 * c + 2, :] = jnp.concatenate(
            [oc[:, 0:_NC // 2], oc[:, _NC // 2:_NC]], axis=0)


@functools.partial(jax.jit, static_argnames=())
def kernel(s1_data, in_lai, in_mask_lai, glob, W1, b1, Wg, W2, b2):
    w1p = jnp.transpose(W1, (3, 0, 2, 1))
    zs1 = jnp.zeros((3, _HID, 3, 4), jnp.float32)
    zt = jnp.zeros((3, _HID, 3, 2), jnp.float32)
    w1m = jnp.concatenate([
        jnp.concatenate([w1p[:, :, :, 0:_CS1], zs1],
                        axis=3).reshape(3, _HID, 48),
        jnp.concatenate([w1p[:, :, :, _CS1:_CS1 + _T], zt],
                        axis=3).reshape(3, _HID, 24),
        jnp.concatenate([w1p[:, :, :, _CS1 + _T:_CIN], zt],
                        axis=3).reshape(3, _HID, 24),
    ], axis=2).reshape(48, 96)
    w2m = jnp.transpose(W2[0], (2, 1, 0)).reshape(3, 48)

    out = pl.pallas_call(
        _conv_kernel,
        grid=(_B,),
        in_specs=[
            pl.BlockSpec((1, _CS1, _NP), lambda b: (b, 0, 0)),
            pl.BlockSpec((1, _T, _NP), lambda b: (b, 0, 0)),
            pl.BlockSpec((1, _T, _NP), lambda b: (b, 0, 0)),
            pl.BlockSpec((1, 1, _G), lambda b: (b, 0, 0)),
            pl.BlockSpec((_G, _HID), lambda b: (0, 0)),
            pl.BlockSpec((1, _HID), lambda b: (0, 0)),
            pl.BlockSpec((48, 96), lambda b: (0, 0)),
            pl.BlockSpec((3, 48), lambda b: (0, 0)),
            pl.BlockSpec((1, 1), lambda b: (0, 0)),
        ],
        out_specs=pl.BlockSpec((1, 8, _NP // 8), lambda b: (b, 0, 0)),
        out_shape=jax.ShapeDtypeStruct((_B, 8, _NP // 8), jnp.float32),
        scratch_shapes=[pltpu.VMEM((_HID, _NF), jnp.float32)],
    )(s1_data.reshape(_B, _CS1, _NP), in_lai.reshape(_B, _T, _NP),
      in_mask_lai.reshape(_B, _T, _NP), glob.reshape(_B, 1, _G), Wg,
      b1.reshape(1, _HID), w1m, w2m, b2.reshape(1, 1))

    return (out.reshape(_B, 1, _H, _W),)


# speedup vs baseline: 41.8259x; 1.9382x over previous
import functools

import jax
import jax.numpy as jnp
from jax import lax
from jax.experimental import pallas as pl
from jax.experimental.pallas import tpu as pltpu

_B, _T, _CS1, _G, _H, _W = 16, 6, 12, 8, 256, 256
_HID = 16
_CIN = _CS1 + 2 * _T

_NP = _H * _W
_NF = (_H + 2) * _W
_NC = 16384
_NCHUNKS = _NP // _NC


def _dyslice(ref, nch, base, dy):
    lo = base + (dy - 1) * _W
    if lo < 0:
        return jnp.concatenate(
            [jnp.zeros((nch, -lo), jnp.float32), ref[:, 0:_NC + lo]],
            axis=1)
    if lo + _NC > _NP:
        return jnp.concatenate(
            [ref[:, lo:_NP], jnp.zeros((nch, lo + _NC - _NP), jnp.float32)],
            axis=1)
    return ref[:, lo:lo + _NC]


def _conv_kernel(s1_ref, lai_ref, msk_ref, glob_ref, wg_ref, b1_ref,
                 w1_ref, w2_ref, b2_ref, out_ref, hz_ref, xs_ref):
    for gg in range(_H // 8):
        blk = jnp.concatenate(
            [s1_ref[0, :, 8 * gg: 8 * gg + 8, :],
             lai_ref[0, :, 8 * gg: 8 * gg + 8, :],
             msk_ref[0, :, 8 * gg: 8 * gg + 8, :]], axis=0)
        xs_ref[:, 2048 * gg: 2048 * (gg + 1)] = blk.reshape(_CIN, 8 * _W)
    g = jnp.dot(glob_ref[0], wg_ref[...],
                preferred_element_type=jnp.float32)
    bias_col = jnp.transpose(g + b1_ref[...], (1, 0))

    lane = lax.broadcasted_iota(jnp.int32, (1, _NC), 1) % _W
    mask_l = lane == 0
    mask_r = lane == _W - 1

    hz_ref[:, 0:_W] = jnp.zeros((_HID, _W), jnp.float32)
    hz_ref[:, _NF - _W:_NF] = jnp.zeros((_HID, _W), jnp.float32)

    zpad4 = jnp.zeros((4, _NC), jnp.float32)
    zpad2 = jnp.zeros((2, _NC), jnp.float32)

    for c in range(_NCHUNKS):
        base = c * _NC
        a3 = jnp.concatenate(
            [_dyslice(xs_ref, _CIN, base, dy) for dy in range(3)],
            axis=0)
        p = lax.dot_general(w1_ref[...], a3, (((1,), (0,)), ((), ())),
                            preferred_element_type=jnp.float32)
        p0 = jnp.where(mask_l, 0.0, jnp.roll(p[0:16], 1, axis=1))
        p2 = jnp.where(mask_r, 0.0, jnp.roll(p[32:48], -1, axis=1))
        hc = p[16:32] + p0 + p2
        hc = jnp.maximum(hc + bias_col, 0.0)
        hz_ref[:, _W + base: _W + base + _NC] = hc

    for c in range(_NCHUNKS):
        base = c * _NC
        h3 = jnp.concatenate(
            [hz_ref[:, base + dy * _W: base + dy * _W + _NC]
             for dy in range(3)], axis=0)
        q = lax.dot_general(w2_ref[...], h3, (((1,), (0,)), ((), ())),
                            preferred_element_type=jnp.float32)
        q0 = jnp.where(mask_l, 0.0, jnp.roll(q[0:1], 1, axis=1))
        q2 = jnp.where(mask_r, 0.0, jnp.roll(q[2:3], -1, axis=1))
        oc = q[1:2] + q0 + q2 + b2_ref[0, 0]
        out_ref[0, 2 * c: 2 * c + 2, :] = jnp.concatenate(
            [oc[:, 0:_NC // 2], oc[:, _NC // 2:_NC]], axis=0)


@functools.partial(jax.jit, static_argnames=())
def kernel(s1_data, in_lai, in_mask_lai, glob, W1, b1, Wg, W2, b2):
    w1m = jnp.transpose(W1, (3, 0, 2, 1)).reshape(48, 72)
    w2m = jnp.transpose(W2[0], (2, 1, 0)).reshape(3, 48)

    out = pl.pallas_call(
        _conv_kernel,
        grid=(_B,),
        in_specs=[
            pl.BlockSpec((1, _CS1, _H, _W), lambda b: (b, 0, 0, 0)),
            pl.BlockSpec((1, _T, _H, _W), lambda b: (b, 0, 0, 0)),
            pl.BlockSpec((1, _T, _H, _W), lambda b: (b, 0, 0, 0)),
            pl.BlockSpec((1, 1, _G), lambda b: (b, 0, 0)),
            pl.BlockSpec((_G, _HID), lambda b: (0, 0)),
            pl.BlockSpec((1, _HID), lambda b: (0, 0)),
            pl.BlockSpec((48, 72), lambda b: (0, 0)),
            pl.BlockSpec((3, 48), lambda b: (0, 0)),
            pl.BlockSpec((1, 1), lambda b: (0, 0)),
        ],
        out_specs=pl.BlockSpec((1, 8, _NP // 8), lambda b: (b, 0, 0)),
        out_shape=jax.ShapeDtypeStruct((_B, 8, _NP // 8), jnp.float32),
        scratch_shapes=[pltpu.VMEM((_HID, _NF), jnp.float32),
                        pltpu.VMEM((_CIN, _NP), jnp.float32)],
    )(s1_data, in_lai, in_mask_lai, glob.reshape(_B, 1, _G), Wg,
      b1.reshape(1, _HID), w1m, w2m, b2.reshape(1, 1))

    return (out.reshape(_B, 1, _H, _W),)
